# baseline (speedup 1.0000x reference)
import functools

import jax
import jax.numpy as jnp
from jax import lax
from jax.experimental import pallas as pl
from jax.experimental.pallas import tpu as pltpu
from jax.experimental.pallas import tpu_sc as plsc

_D = 64
_SCALE = 8.0

_NC = 2
_NS = 16
_NW = _NC * _NS

_CHUNK = 128


@functools.lru_cache(maxsize=None)
def _emb_fn(B):
    b_per_w = B // _NW
    n_chunks = b_per_w // _CHUNK
    mesh = plsc.VectorSubcoreMesh(core_axis_name="c", subcore_axis_name="s")

    @functools.partial(
        pl.kernel,
        mesh=mesh,
        compiler_params=pltpu.CompilerParams(use_tc_tiling_on_sc=False),
        out_type=jax.ShapeDtypeStruct((B, _D), jnp.float32),
        scratch_types=[
            pltpu.VMEM((b_per_w,), jnp.int32),
            pltpu.VMEM((_CHUNK, _D), jnp.float32),
            pltpu.SemaphoreType.DMA,
        ],
    )
    def emb(table_hbm, idx_hbm, out_hbm, idx_v, rows_v, sem):
        wid = lax.axis_index("s") * _NC + lax.axis_index("c")
        base = wid * b_per_w
        pltpu.sync_copy(idx_hbm.at[pl.ds(base, b_per_w)], idx_v)

        def chunk_body(c, carry):
            off = c * _CHUNK
            pltpu.async_copy(
                table_hbm.at[idx_v.at[pl.ds(off, _CHUNK)]], rows_v, sem
            ).wait()

            def row_body(r, rcarry):
                for k in range(_D // 16):
                    sl = pl.ds(k * 16, 16)
                    rows_v[r, sl] = rows_v[r, sl] * _SCALE
                return rcarry

            lax.fori_loop(0, _CHUNK, row_body, 0)
            pltpu.sync_copy(rows_v, out_hbm.at[pl.ds(base + off, _CHUNK)])
            return carry

        lax.fori_loop(0, n_chunks, chunk_body, 0)

    return emb


def kernel(x, embedding_weight):
    s0, s1 = x.shape
    B = s0 * s1
    idx = x.reshape(B).astype(jnp.int32)
    out = _emb_fn(B)(embedding_weight, idx)
    return out.reshape(s0, s1, _D)


# speedup vs baseline: 1.1139x; 1.1139x over previous
import functools

import jax
import jax.numpy as jnp
from jax import lax
from jax.experimental import pallas as pl
from jax.experimental.pallas import tpu as pltpu
from jax.experimental.pallas import tpu_sc as plsc

_D = 64
_SCALE = 8.0

_NC = 2
_NS = 16
_NW = _NC * _NS

_CHUNK = 128
_NBUF = 8
_UNROLL = 8


def _scale_buf(buf):
    def body(i, carry):
        r0 = i * _UNROLL
        for dr in range(_UNROLL):
            for k in range(_D // 16):
                sl = pl.ds(k * 16, 16)
                buf[r0 + dr, sl] = buf[r0 + dr, sl] * _SCALE
        return carry

    lax.fori_loop(0, _CHUNK // _UNROLL, body, 0)


@functools.lru_cache(maxsize=None)
def _emb_fn(B):
    b_per_w = B // _NW
    n_chunks = b_per_w // _CHUNK
    n_groups = n_chunks // _NBUF
    mesh = plsc.VectorSubcoreMesh(core_axis_name="c", subcore_axis_name="s")

    scratch = [pltpu.VMEM((b_per_w,), jnp.int32)]
    scratch += [pltpu.VMEM((_CHUNK, _D), jnp.float32) for _ in range(_NBUF)]
    scratch += [pltpu.SemaphoreType.DMA for _ in range(2 * _NBUF)]

    @functools.partial(
        pl.kernel,
        mesh=mesh,
        compiler_params=pltpu.CompilerParams(use_tc_tiling_on_sc=False),
        out_type=jax.ShapeDtypeStruct((B, _D), jnp.float32),
        scratch_types=scratch,
    )
    def emb(table_hbm, idx_hbm, out_hbm, idx_v, *rest):
        bufs = rest[:_NBUF]
        gsem = rest[_NBUF:2 * _NBUF]
        osem = rest[2 * _NBUF:]

        wid = lax.axis_index("s") * _NC + lax.axis_index("c")
        base = wid * b_per_w
        pltpu.sync_copy(idx_hbm.at[pl.ds(base, b_per_w)], idx_v)

        def group_body(g, carry):
            c0 = g * (_NBUF * _CHUNK)
            for b in range(_NBUF):
                off = c0 + b * _CHUNK
                dst = out_hbm.at[pl.ds(base + off, _CHUNK)]

                @pl.when(g != 0)
                def _drain():
                    pltpu.make_async_copy(bufs[b], dst, osem[b]).wait()

                pltpu.async_copy(
                    table_hbm.at[idx_v.at[pl.ds(off, _CHUNK)]], bufs[b], gsem[b]
                )
            for b in range(_NBUF):
                off = c0 + b * _CHUNK
                dst = out_hbm.at[pl.ds(base + off, _CHUNK)]
                pltpu.make_async_copy(
                    table_hbm.at[idx_v.at[pl.ds(off, _CHUNK)]], bufs[b], gsem[b]
                ).wait()
                _scale_buf(bufs[b])
                pltpu.async_copy(bufs[b], dst, osem[b])
            return carry

        lax.fori_loop(0, n_groups, group_body, 0)
        for b in range(_NBUF):
            off = (n_groups - 1) * (_NBUF * _CHUNK) + b * _CHUNK
            dst = out_hbm.at[pl.ds(base + off, _CHUNK)]
            pltpu.make_async_copy(bufs[b], dst, osem[b]).wait()

    return emb


def kernel(x, embedding_weight):
    s0, s1 = x.shape
    B = s0 * s1
    idx = x.reshape(B).astype(jnp.int32)
    out = _emb_fn(B)(embedding_weight, idx)
    return out.reshape(s0, s1, _D)
